# baseline (speedup 1.0000x reference)
import functools

import jax
import jax.numpy as jnp
from jax import lax
from jax.experimental import pallas as pl
from jax.experimental.pallas import tpu as pltpu
from jax.experimental.pallas import tpu_sc as plsc

CLS = 100000
FEAT = 64
BATCH = 16384
NC = 2
NS = 16
NW = NC * NS
BPW = BATCH // NW
IDX_ROWS = BPW // 128
CNT_ROWS = BATCH // NS // 128
CPAD = 100352
ZCHUNK = CPAD // NS


def _body(xs_hbm, ys2_hbm, center2_hbm, out_hbm,
          cidx_v, idx2_v, par_v, zbuf, ones_v, xs_v, rows_v, cnt_v, w_v,
          acc_v, counts_sh, sem_c, sem_x):
    c = lax.axis_index("c")
    s = lax.axis_index("s")
    wid = s * NC + c

    zero16 = jnp.zeros((16,), jnp.float32)
    one16 = jnp.ones((16,), jnp.float32)

    pltpu.sync_copy(ys2_hbm.at[pl.ds(s * CNT_ROWS, CNT_ROWS)], cidx_v)
    crow = c * IDX_ROWS

    for j in range(IDX_ROWS):
        for t in range(128 // 16):
            y16 = cidx_v[crow + j, pl.ds(t * 16, 16)]
            idx2_v[j, pl.ds(t * 16, 16)] = lax.shift_right_logical(y16, 1)
            par_v[pl.ds(j * 128 + t * 16, 16)] = (y16 & 1) * FEAT

    cp_x = pltpu.async_copy(
        xs_hbm.at[pl.ds(s * (BPW * NC // 2) + c * (BPW // 2), BPW // 2)],
        xs_v, sem_x)
    cps = [
        pltpu.async_copy(center2_hbm.at[idx2_v.at[j]],
                         rows_v.at[pl.ds(j * 128, 128)], sem_c)
        for j in range(IDX_ROWS)
    ]

    def zb(i, carry):
        zbuf[pl.ds(i * 16, 16)] = zero16
        return carry

    lax.fori_loop(0, ZCHUNK // 16, zb, 0)
    for t in range(128 // 16):
        ones_v[pl.ds(t * 16, 16)] = one16

    pltpu.sync_copy(zbuf, counts_sh.at[pl.ds(s * ZCHUNK, ZCHUNK)])
    plsc.subcore_barrier()
    for j in range(CNT_ROWS):
        pltpu.sync_copy(ones_v, counts_sh.at[cidx_v.at[j]], add=True)
    plsc.subcore_barrier()

    for j in range(IDX_ROWS):
        pltpu.sync_copy(counts_sh.at[cidx_v.at[crow + j]], cnt_v.at[j])
    for k in range(BPW // 16):
        occ = cnt_v[k // 8, pl.ds((k % 8) * 16, 16)]
        w_v[pl.ds(k * 16, 16)] = 0.5 / (occ + 1.0)

    cp_x.wait()
    for cp in cps:
        cp.wait()

    def body(g, acc):
        wch = w_v[pl.ds(g * 16, 16)]
        pch = par_v[pl.ds(g * 16, 16)]
        for lane in range(16):
            wi = wch[lane]
            off = pch[lane]
            i = g * 16 + lane
            xrow = g * 8 + lane // 2
            xcol = (lane % 2) * FEAT
            for k in range(FEAT // 16):
                x16 = xs_v[xrow, pl.ds(xcol + k * 16, 16)]
                c16 = rows_v[i, pl.ds(off + k * 16, 16)]
                d = x16 - c16
                acc = acc + d * d * wi
        return acc

    acc = lax.fori_loop(0, BPW // 16, body, jnp.zeros((16,), jnp.float32))
    acc_v[...] = acc
    pltpu.sync_copy(acc_v, out_hbm.at[pl.ds(wid * 16, 16)])


def kernel(xs, ys, center):
    xs2 = xs.reshape(BATCH // 2, 2 * FEAT)
    ys2 = ys.astype(jnp.int32).reshape(128, 128)
    center2 = center.reshape(CLS // 2, 2 * FEAT)
    mesh = plsc.VectorSubcoreMesh(core_axis_name="c", subcore_axis_name="s")
    k = pl.kernel(
        _body,
        mesh=mesh,
        out_type=jax.ShapeDtypeStruct((NW * 16,), jnp.float32),
        scratch_types=[
            pltpu.VMEM((CNT_ROWS, 128), jnp.int32),
            pltpu.VMEM((IDX_ROWS, 128), jnp.int32),
            pltpu.VMEM((BPW,), jnp.int32),
            pltpu.VMEM((ZCHUNK,), jnp.float32),
            pltpu.VMEM((128,), jnp.float32),
            pltpu.VMEM((BPW // 2, 2 * FEAT), jnp.float32),
            pltpu.VMEM((BPW, 2 * FEAT), jnp.float32),
            pltpu.VMEM((IDX_ROWS, 128), jnp.float32),
            pltpu.VMEM((BPW,), jnp.float32),
            pltpu.VMEM((16,), jnp.float32),
            pltpu.VMEM_SHARED((CPAD,), jnp.float32),
            pltpu.SemaphoreType.DMA,
            pltpu.SemaphoreType.DMA,
        ],
    )
    out = k(xs2, ys2, center2)
    return jnp.sum(out)


# speedup vs baseline: 1.0027x; 1.0027x over previous
import functools

import jax
import jax.numpy as jnp
from jax import lax
from jax.experimental import pallas as pl
from jax.experimental.pallas import tpu as pltpu
from jax.experimental.pallas import tpu_sc as plsc

CLS = 100000
FEAT = 64
BATCH = 16384
NC = 2
NS = 16
NW = NC * NS
BPW = BATCH // NW
IDX_ROWS = BPW // 128
CNT_ROWS = BATCH // NS // 128
CPAD = 100352
ZCHUNK = CPAD // NS
ZBUF = 1568


def _body(xs_hbm, ys2_hbm, center2_hbm, out_hbm,
          cidx_v, idx2_v, par_v, zbuf, ones_v, xs_v, rows_a, rows_b,
          cnt_v, w_v, acc_v, counts_sh, sem_a, sem_b, sem_x):
    c = lax.axis_index("c")
    s = lax.axis_index("s")
    wid = s * NC + c

    zero16 = jnp.zeros((16,), jnp.float32)
    one16 = jnp.ones((16,), jnp.float32)

    pltpu.sync_copy(ys2_hbm.at[pl.ds(s * CNT_ROWS, CNT_ROWS)], cidx_v)
    crow = c * IDX_ROWS

    for j in range(IDX_ROWS):
        for t in range(128 // 16):
            y16 = cidx_v[crow + j, pl.ds(t * 16, 16)]
            idx2_v[j, pl.ds(t * 16, 16)] = lax.shift_right_logical(y16, 1)
            par_v[pl.ds(j * 128 + t * 16, 16)] = (y16 & 1) * FEAT

    cp_x = pltpu.async_copy(
        xs_hbm.at[pl.ds(s * (2 * BPW // 8) + c * (BPW // 8), BPW // 8)],
        xs_v, sem_x)

    def zb(i, carry):
        zbuf[pl.ds(i * 16, 16)] = zero16
        return carry

    lax.fori_loop(0, ZBUF // 16, zb, 0)
    for t in range(128 // 16):
        ones_v[pl.ds(t * 16, 16)] = one16

    for q in range(ZCHUNK // ZBUF):
        pltpu.sync_copy(zbuf, counts_sh.at[pl.ds(s * ZCHUNK + q * ZBUF, ZBUF)])
    plsc.subcore_barrier()
    for j in range(CNT_ROWS):
        pltpu.sync_copy(ones_v, counts_sh.at[cidx_v.at[j]], add=True)
    plsc.subcore_barrier()

    for j in range(IDX_ROWS):
        pltpu.sync_copy(counts_sh.at[cidx_v.at[crow + j]], cnt_v.at[j])
    for k in range(BPW // 16):
        occ = cnt_v[k // 8, pl.ds((k % 8) * 16, 16)]
        w_v[pl.ds(k * 16, 16)] = 0.5 / (occ + 1.0)

    cp_x.wait()

    bufs = (rows_a, rows_b)
    sems = (sem_a, sem_b)

    def fire(ch):
        return pltpu.async_copy(center2_hbm.at[idx2_v.at[ch]],
                                bufs[ch % 2], sems[ch % 2])

    def group_body(ch, buf):
        def inner(gg, acc):
            g = ch * 8 + gg
            wch = w_v[pl.ds(g * 16, 16)]
            pch = par_v[pl.ds(g * 16, 16)]
            for lane in range(16):
                wi = wch[lane]
                off = pch[lane]
                grp = ch * 16 + gg * 2 + lane // 8
                for k in range(FEAT // 16):
                    x16 = xs_v[grp, lane % 8, pl.ds(k * 16, 16)]
                    c16 = buf[gg * 16 + lane, pl.ds(off + k * 16, 16)]
                    d = x16 - c16
                    acc = acc + d * d * wi
            return acc
        return inner

    acc = jnp.zeros((16,), jnp.float32)
    cur = fire(0)
    for ch in range(IDX_ROWS):
        nxt = fire(ch + 1) if ch + 1 < IDX_ROWS else None
        cur.wait()
        acc = lax.fori_loop(0, 8, group_body(ch, bufs[ch % 2]), acc)
        cur = nxt

    acc_v[...] = acc
    pltpu.sync_copy(acc_v, out_hbm.at[pl.ds(wid * 16, 16)])


def kernel(xs, ys, center):
    xs3 = xs.reshape(BATCH // 8, 8, FEAT)
    ys2 = ys.astype(jnp.int32).reshape(128, 128)
    center2 = center.reshape(CLS // 2, 2 * FEAT)
    mesh = plsc.VectorSubcoreMesh(core_axis_name="c", subcore_axis_name="s")
    k = pl.kernel(
        _body,
        mesh=mesh,
        out_type=jax.ShapeDtypeStruct((NW * 16,), jnp.float32),
        scratch_types=[
            pltpu.VMEM((CNT_ROWS, 128), jnp.int32),
            pltpu.VMEM((IDX_ROWS, 128), jnp.int32),
            pltpu.VMEM((BPW,), jnp.int32),
            pltpu.VMEM((ZBUF,), jnp.float32),
            pltpu.VMEM((128,), jnp.float32),
            pltpu.VMEM((BPW // 8, 8, FEAT), jnp.float32),
            pltpu.VMEM((128, 2 * FEAT), jnp.float32),
            pltpu.VMEM((128, 2 * FEAT), jnp.float32),
            pltpu.VMEM((IDX_ROWS, 128), jnp.float32),
            pltpu.VMEM((BPW,), jnp.float32),
            pltpu.VMEM((16,), jnp.float32),
            pltpu.VMEM_SHARED((CPAD,), jnp.float32),
            pltpu.SemaphoreType.DMA,
            pltpu.SemaphoreType.DMA,
            pltpu.SemaphoreType.DMA,
        ],
    )
    out = k(xs3, ys2, center2)
    return jnp.sum(out)


# speedup vs baseline: 1.0373x; 1.0346x over previous
import functools

import jax
import jax.numpy as jnp
from jax import lax
from jax.experimental import pallas as pl
from jax.experimental.pallas import tpu as pltpu
from jax.experimental.pallas import tpu_sc as plsc

CLS = 100000
FEAT = 64
BATCH = 16384
NC = 2
NS = 16
NW = NC * NS
BPW = BATCH // NW
IDX_ROWS = BPW // 128
CNT_ROWS = BATCH // NS // 128
CPAD = 100352
ZCHUNK = CPAD // NS


def _body(xs_hbm, ys2_hbm, center_hbm, out_hbm,
          cidx_v, idx_v, zbuf, ones_v, xs_v, rows_v, cnt_v, w_v, acc_v,
          counts_sh, sem_c, sem_x):
    c = lax.axis_index("c")
    s = lax.axis_index("s")
    wid = s * NC + c

    zero16 = jnp.zeros((16,), jnp.float32)
    one16 = jnp.ones((16,), jnp.float32)

    pltpu.sync_copy(ys2_hbm.at[pl.ds(wid * IDX_ROWS, IDX_ROWS)], idx_v)
    cp_x = pltpu.async_copy(xs_hbm.at[pl.ds(wid * BPW, BPW)], xs_v, sem_x)
    cps = [
        pltpu.async_copy(center_hbm.at[idx_v.at[j]],
                         rows_v.at[pl.ds(j * 128, 128)], sem_c)
        for j in range(IDX_ROWS)
    ]

    pltpu.sync_copy(ys2_hbm.at[pl.ds(s * CNT_ROWS, CNT_ROWS)], cidx_v)

    def zb(i, carry):
        zbuf[pl.ds(i * 16, 16)] = zero16
        return carry

    lax.fori_loop(0, ZCHUNK // 16, zb, 0)
    for t in range(128 // 16):
        ones_v[pl.ds(t * 16, 16)] = one16

    pltpu.sync_copy(zbuf, counts_sh.at[pl.ds(s * ZCHUNK, ZCHUNK)])
    plsc.subcore_barrier()
    for j in range(CNT_ROWS):
        pltpu.sync_copy(ones_v, counts_sh.at[cidx_v.at[j]], add=True)
    plsc.subcore_barrier()

    for j in range(IDX_ROWS):
        pltpu.sync_copy(counts_sh.at[idx_v.at[j]], cnt_v.at[j])
    for k in range(BPW // 16):
        occ = cnt_v[k // 8, pl.ds((k % 8) * 16, 16)]
        w_v[pl.ds(k * 16, 16)] = 0.5 / (occ + 1.0)

    cp_x.wait()
    for cp in cps:
        cp.wait()

    def body(g, acc):
        wch = w_v[pl.ds(g * 16, 16)]
        base = g * 16
        for lane in range(16):
            wi = wch[lane]
            i = base + lane
            for k in range(FEAT // 16):
                x16 = xs_v[i, pl.ds(k * 16, 16)]
                c16 = rows_v[i, pl.ds(k * 16, 16)]
                d = x16 - c16
                acc = acc + d * d * wi
        return acc

    acc = lax.fori_loop(0, BPW // 16, body, jnp.zeros((16,), jnp.float32))
    acc_v[...] = acc
    pltpu.sync_copy(acc_v, out_hbm.at[wid])


def kernel(xs, ys, center):
    ys2 = ys.astype(jnp.int32).reshape(128, 128)
    mesh = plsc.VectorSubcoreMesh(core_axis_name="c", subcore_axis_name="s")
    k = pl.kernel(
        _body,
        mesh=mesh,
        compiler_params=pltpu.CompilerParams(use_tc_tiling_on_sc=False),
        out_type=jax.ShapeDtypeStruct((NW, 16), jnp.float32),
        scratch_types=[
            pltpu.VMEM((CNT_ROWS, 128), jnp.int32),
            pltpu.VMEM((IDX_ROWS, 128), jnp.int32),
            pltpu.VMEM((ZCHUNK,), jnp.float32),
            pltpu.VMEM((128,), jnp.float32),
            pltpu.VMEM((BPW, FEAT), jnp.float32),
            pltpu.VMEM((BPW, FEAT), jnp.float32),
            pltpu.VMEM((IDX_ROWS, 128), jnp.float32),
            pltpu.VMEM((BPW,), jnp.float32),
            pltpu.VMEM((16,), jnp.float32),
            pltpu.VMEM_SHARED((CPAD,), jnp.float32),
            pltpu.SemaphoreType.DMA,
            pltpu.SemaphoreType.DMA,
        ],
    )
    out = k(xs, ys2, center)
    return jnp.sum(out)
